# baseline (speedup 1.0000x reference)
import functools

import jax
import jax.numpy as jnp
from jax import lax
from jax.experimental import pallas as pl
from jax.experimental.pallas import tpu as pltpu
from jax.experimental.pallas import tpu_sc as plsc

N = 10000
D = 128
E = 320000
NC = 2
NS = 16
NW = NC * NS
EPW = E // NW
CHUNK = 80
NCHUNK = EPW // CHUNK
NPAD = 10240
RPT = NPAD // NS
NBUF = 4

_mesh = plsc.VectorSubcoreMesh(core_axis_name="c", subcore_axis_name="s")


@functools.partial(
    pl.kernel,
    out_type=jax.ShapeDtypeStruct((NC, NPAD, D), jnp.float32),
    mesh=_mesh,
    scratch_types=[
        pltpu.VMEM((NCHUNK, CHUNK), jnp.int32),
        pltpu.VMEM((CHUNK, D), jnp.float32),
        pltpu.VMEM_SHARED((NPAD, D), jnp.float32),
        pltpu.SemaphoreType.DMA,
        pltpu.SemaphoreType.DMA,
    ],
)
def _sc_degree(dst_hbm, ones_hbm, zeros_hbm, out_hbm, didx_v, ones_v,
               acc_sh, sem0, sem1):
    c = lax.axis_index("c")
    s = lax.axis_index("s")
    wid = c * NS + s
    pltpu.sync_copy(ones_hbm, ones_v)
    pltpu.sync_copy(zeros_hbm.at[pl.ds(s * RPT, RPT)],
                    acc_sh.at[pl.ds(s * RPT, RPT)])
    pltpu.sync_copy(dst_hbm.at[wid], didx_v)
    plsc.subcore_barrier()

    sems = (sem0, sem1)
    for b in range(2):
        pltpu.async_copy(ones_v, acc_sh.at[didx_v.at[b]], sems[b], add=True)

    def grp(g, carry):
        for b in range(2):
            i = g * 2 + b
            pltpu.make_async_copy(ones_v, acc_sh.at[didx_v.at[i]],
                                  sems[b]).wait()

            @pl.when(i + 2 < NCHUNK)
            def _():
                pltpu.async_copy(ones_v, acc_sh.at[didx_v.at[i + 2]],
                                 sems[b], add=True)
        return carry

    lax.fori_loop(0, NCHUNK // 2, grp, 0)
    pltpu.make_async_copy(ones_v, acc_sh.at[didx_v.at[NCHUNK - 1]],
                          sems[(NCHUNK - 1) % 2]).wait()
    plsc.subcore_barrier()
    pltpu.sync_copy(acc_sh.at[pl.ds(s * RPT, RPT)],
                    out_hbm.at[c, pl.ds(s * RPT, RPT)])


@functools.partial(
    pl.kernel,
    out_type=jax.ShapeDtypeStruct((NC, NPAD, D), jnp.float32),
    mesh=_mesh,
    scratch_types=[
        pltpu.VMEM((2 * NBUF, CHUNK), jnp.int32),
        pltpu.VMEM((NBUF, CHUNK, D), jnp.float32),
        pltpu.VMEM_SHARED((NPAD, D), jnp.float32),
        pltpu.SemaphoreType.DMA,
        pltpu.SemaphoreType.DMA,
        pltpu.SemaphoreType.DMA,
        pltpu.SemaphoreType.DMA,
        pltpu.SemaphoreType.DMA,
        pltpu.SemaphoreType.DMA,
        pltpu.SemaphoreType.DMA,
        pltpu.SemaphoreType.DMA,
    ],
)
def _sc_scatter(g_hbm, src_hbm, dst_hbm, zeros_hbm, out_hbm,
                idx_v, rows_v, acc_sh,
                isem0, isem1, isem2, isem3, gsem0, gsem1, gsem2, gsem3):
    c = lax.axis_index("c")
    s = lax.axis_index("s")
    wid = c * NS + s
    isems = (isem0, isem1, isem2, isem3)
    gsems = (gsem0, gsem1, gsem2, gsem3)
    ebase = wid * EPW

    pltpu.sync_copy(zeros_hbm.at[pl.ds(s * RPT, RPT)],
                    acc_sh.at[pl.ds(s * RPT, RPT)])
    plsc.subcore_barrier()

    def fire_idx(j, b):
        off = ebase + j * CHUNK
        pltpu.async_copy(src_hbm.at[pl.ds(off, CHUNK)], idx_v.at[2 * b],
                         isems[b])
        pltpu.async_copy(dst_hbm.at[pl.ds(off, CHUNK)], idx_v.at[2 * b + 1],
                         isems[b])

    def wait_idx(b):
        pltpu.make_async_copy(src_hbm.at[pl.ds(0, CHUNK)], idx_v.at[2 * b],
                              isems[b]).wait()
        pltpu.make_async_copy(src_hbm.at[pl.ds(0, CHUNK)],
                              idx_v.at[2 * b + 1], isems[b]).wait()

    def fire_gather(b):
        pltpu.async_copy(g_hbm.at[idx_v.at[2 * b]], rows_v.at[b], gsems[b])

    def wait_gather(b):
        pltpu.make_async_copy(g_hbm.at[idx_v.at[2 * b]], rows_v.at[b],
                              gsems[b]).wait()

    for j in range(NBUF - 1):
        fire_idx(j, j)
    for j in range(NBUF - 2):
        wait_idx(j)
        fire_gather(j)

    def grp(g, carry):
        for b in range(NBUF):
            i = g * NBUF + b

            wait_gather(b)
            pltpu.sync_copy(rows_v.at[b], acc_sh.at[idx_v.at[2 * b + 1]],
                            add=True)

            @pl.when(i + NBUF - 1 < NCHUNK)
            def _():
                fire_idx(i + NBUF - 1, (b + NBUF - 1) % NBUF)

            @pl.when(i + NBUF - 2 < NCHUNK)
            def _():
                nb = (b + NBUF - 2) % NBUF
                wait_idx(nb)
                fire_gather(nb)
        return carry

    lax.fori_loop(0, NCHUNK // NBUF, grp, 0)
    tb = (NCHUNK - 1) % NBUF
    wait_gather(tb)
    pltpu.sync_copy(rows_v.at[tb], acc_sh.at[idx_v.at[2 * tb + 1]], add=True)

    plsc.subcore_barrier()
    pltpu.sync_copy(acc_sh.at[pl.ds(s * RPT, RPT)],
                    out_hbm.at[c, pl.ds(s * RPT, RPT)])


def _dinv_from_parts(degp):
    deg = degp[0, :N, 0:1] + degp[1, :N, 0:1] + 1.0
    return lax.rsqrt(deg)


def _tc_matmul_body(x_ref, w1_ref, h_ref):
    h_ref[...] = jnp.dot(x_ref[...], w1_ref[...],
                         preferred_element_type=jnp.float32)


def _tc_scale_body(h_ref, degp_ref, g_ref):
    g_ref[...] = h_ref[...] * _dinv_from_parts(degp_ref[...])


def _tc_mid_body(acc_ref, g1_ref, degp_ref, b1_ref, gamma_ref, beta_ref,
                 w2_ref, g2_ref):
    dinv = _dinv_from_parts(degp_ref[...])
    acc = acc_ref[:, :N, :]
    h1 = dinv * (acc[0] + acc[1] + g1_ref[...]) + b1_ref[...]
    mean = jnp.mean(h1, axis=0, keepdims=True)
    var = jnp.mean((h1 - mean) ** 2, axis=0, keepdims=True)
    hn = gamma_ref[...] * (h1 - mean) * lax.rsqrt(var + 1e-5) + beta_ref[...]
    hn = jnp.where(hn > 0, hn, 0.01 * hn)
    h2 = jnp.dot(hn, w2_ref[...], preferred_element_type=jnp.float32)
    g2_ref[...] = h2 * dinv


def _tc_back_body(acc_ref, g2_ref, degp_ref, b2_ref, out_ref):
    dinv = _dinv_from_parts(degp_ref[...])
    acc = acc_ref[:, :N, :]
    out_ref[...] = dinv * (acc[0] + acc[1] + g2_ref[...]) + b2_ref[...]


def kernel(x, edge_index, W1, b1, gamma, beta, W2, b2):
    src = edge_index[0]
    dst = edge_index[1]
    dst2 = dst.reshape(NW, NCHUNK, CHUNK)
    zeros = jnp.zeros((NPAD, D), jnp.float32)
    ones_rows = jnp.ones((CHUNK, D), jnp.float32)
    deg_parts = _sc_degree(dst2, ones_rows, zeros)
    h = pl.pallas_call(
        _tc_matmul_body,
        out_shape=jax.ShapeDtypeStruct((N, D), jnp.float32),
    )(x, W1)
    g1 = pl.pallas_call(
        _tc_scale_body,
        out_shape=jax.ShapeDtypeStruct((N, D), jnp.float32),
    )(h, deg_parts)
    acc1 = _sc_scatter(g1, src, dst, zeros)
    g2 = pl.pallas_call(
        _tc_mid_body,
        out_shape=jax.ShapeDtypeStruct((N, D), jnp.float32),
    )(acc1, g1, deg_parts, b1.reshape(1, D), gamma.reshape(1, D),
      beta.reshape(1, D), W2)
    acc2 = _sc_scatter(g2, src, dst, zeros)
    out = pl.pallas_call(
        _tc_back_body,
        out_shape=jax.ShapeDtypeStruct((N, D), jnp.float32),
    )(acc2, g2, deg_parts, b2.reshape(1, D))
    return out


# speedup vs baseline: 1.0117x; 1.0117x over previous
import functools

import jax
import jax.numpy as jnp
from jax import lax
from jax.experimental import pallas as pl
from jax.experimental.pallas import tpu as pltpu
from jax.experimental.pallas import tpu_sc as plsc

N = 10000
D = 128
E = 320000
NC = 2
NS = 16
NW = NC * NS
EPW = E // NW
CHUNK = 80
NCHUNK = EPW // CHUNK
NPAD = 10240
RPT = NPAD // NS
NBUF = 4

_mesh = plsc.VectorSubcoreMesh(core_axis_name="c", subcore_axis_name="s")


@functools.partial(
    pl.kernel,
    out_type=jax.ShapeDtypeStruct((NC, NPAD, D), jnp.float32),
    mesh=_mesh,
    scratch_types=[
        pltpu.VMEM((NCHUNK, CHUNK), jnp.int32),
        pltpu.VMEM((CHUNK, D), jnp.float32),
        pltpu.VMEM_SHARED((NPAD, D), jnp.float32),
        pltpu.SemaphoreType.DMA,
        pltpu.SemaphoreType.DMA,
    ],
)
def _sc_degree(dst_hbm, ones_hbm, zeros_hbm, out_hbm, didx_v, ones_v,
               acc_sh, sem0, sem1):
    c = lax.axis_index("c")
    s = lax.axis_index("s")
    wid = c * NS + s
    pltpu.sync_copy(ones_hbm, ones_v)
    pltpu.sync_copy(zeros_hbm.at[pl.ds(s * RPT, RPT)],
                    acc_sh.at[pl.ds(s * RPT, RPT)])
    pltpu.sync_copy(dst_hbm.at[wid], didx_v)
    plsc.subcore_barrier()

    sems = (sem0, sem1)
    for b in range(2):
        pltpu.async_copy(ones_v, acc_sh.at[didx_v.at[b]], sems[b], add=True)

    def grp(g, carry):
        for b in range(2):
            i = g * 2 + b
            pltpu.make_async_copy(ones_v, acc_sh.at[didx_v.at[i]],
                                  sems[b]).wait()

            @pl.when(i + 2 < NCHUNK)
            def _():
                pltpu.async_copy(ones_v, acc_sh.at[didx_v.at[i + 2]],
                                 sems[b], add=True)
        return carry

    lax.fori_loop(0, NCHUNK // 2, grp, 0)
    pltpu.make_async_copy(ones_v, acc_sh.at[didx_v.at[NCHUNK - 1]],
                          sems[(NCHUNK - 1) % 2]).wait()
    plsc.subcore_barrier()
    pltpu.sync_copy(acc_sh.at[pl.ds(s * RPT, RPT)],
                    out_hbm.at[c, pl.ds(s * RPT, RPT)])


@functools.partial(
    pl.kernel,
    out_type=jax.ShapeDtypeStruct((NC, NPAD, D), jnp.float32),
    mesh=_mesh,
    scratch_types=[
        pltpu.VMEM((2 * NBUF, CHUNK), jnp.int32),
        pltpu.VMEM((NBUF, CHUNK, D), jnp.float32),
        pltpu.VMEM_SHARED((NPAD, D), jnp.float32),
        pltpu.SemaphoreType.DMA,
        pltpu.SemaphoreType.DMA,
        pltpu.SemaphoreType.DMA,
        pltpu.SemaphoreType.DMA,
        pltpu.SemaphoreType.DMA,
        pltpu.SemaphoreType.DMA,
        pltpu.SemaphoreType.DMA,
        pltpu.SemaphoreType.DMA,
    ],
)
def _sc_scatter(g_hbm, src_hbm, dst_hbm, zeros_hbm, out_hbm,
                idx_v, rows_v, acc_sh,
                isem0, isem1, isem2, isem3, gsem0, gsem1, gsem2, gsem3):
    c = lax.axis_index("c")
    s = lax.axis_index("s")
    wid = c * NS + s
    isems = (isem0, isem1, isem2, isem3)
    gsems = (gsem0, gsem1, gsem2, gsem3)
    ebase = wid * EPW

    pltpu.sync_copy(zeros_hbm.at[pl.ds(s * RPT, RPT)],
                    acc_sh.at[pl.ds(s * RPT, RPT)])
    plsc.subcore_barrier()

    def fire_idx(j, b):
        off = ebase + j * CHUNK
        pltpu.async_copy(src_hbm.at[pl.ds(off, CHUNK)], idx_v.at[2 * b],
                         isems[b])
        pltpu.async_copy(dst_hbm.at[pl.ds(off, CHUNK)], idx_v.at[2 * b + 1],
                         isems[b])

    def wait_idx(b):
        pltpu.make_async_copy(src_hbm.at[pl.ds(0, CHUNK)], idx_v.at[2 * b],
                              isems[b]).wait()
        pltpu.make_async_copy(src_hbm.at[pl.ds(0, CHUNK)],
                              idx_v.at[2 * b + 1], isems[b]).wait()

    def fire_gather(b):
        pltpu.async_copy(g_hbm.at[idx_v.at[2 * b]], rows_v.at[b], gsems[b])

    def wait_gather(b):
        pltpu.make_async_copy(g_hbm.at[idx_v.at[2 * b]], rows_v.at[b],
                              gsems[b]).wait()

    for j in range(NBUF - 1):
        fire_idx(j, j)
    for j in range(NBUF - 2):
        wait_idx(j)
        fire_gather(j)

    def grp(g, carry):
        for b in range(NBUF):
            i = g * NBUF + b

            wait_gather(b)
            pltpu.sync_copy(rows_v.at[b], acc_sh.at[idx_v.at[2 * b + 1]],
                            add=True)

            @pl.when(i + NBUF - 1 < NCHUNK)
            def _():
                fire_idx(i + NBUF - 1, (b + NBUF - 1) % NBUF)

            @pl.when(i + NBUF - 2 < NCHUNK)
            def _():
                nb = (b + NBUF - 2) % NBUF
                wait_idx(nb)
                fire_gather(nb)
        return carry

    lax.fori_loop(0, NCHUNK // NBUF, grp, 0)
    tb = (NCHUNK - 1) % NBUF
    wait_gather(tb)
    pltpu.sync_copy(rows_v.at[tb], acc_sh.at[idx_v.at[2 * tb + 1]], add=True)

    plsc.subcore_barrier()
    pltpu.sync_copy(acc_sh.at[pl.ds(s * RPT, RPT)],
                    out_hbm.at[c, pl.ds(s * RPT, RPT)])


def _dinv_from_parts(degp):
    deg = degp[0, :N, 0:1] + degp[1, :N, 0:1] + 1.0
    return lax.rsqrt(deg)


def _tc_front_body(x_ref, w1_ref, degp_ref, g_ref, dinv_ref):
    dinv = _dinv_from_parts(degp_ref[...])
    h = jnp.dot(x_ref[...], w1_ref[...], preferred_element_type=jnp.float32)
    g_ref[...] = h * dinv
    dinv_ref[...] = jnp.broadcast_to(dinv, (N, D))


def _tc_mid_body(acc_ref, g1_ref, dinv_ref, b1_ref, gamma_ref, beta_ref,
                 w2_ref, g2_ref):
    dinv = dinv_ref[...]
    acc = acc_ref[:, :N, :]
    h1 = dinv * (acc[0] + acc[1] + g1_ref[...]) + b1_ref[...]
    mean = jnp.mean(h1, axis=0, keepdims=True)
    var = jnp.mean((h1 - mean) ** 2, axis=0, keepdims=True)
    hn = gamma_ref[...] * (h1 - mean) * lax.rsqrt(var + 1e-5) + beta_ref[...]
    hn = jnp.where(hn > 0, hn, 0.01 * hn)
    h2 = jnp.dot(hn, w2_ref[...], preferred_element_type=jnp.float32)
    g2_ref[...] = h2 * dinv


def _tc_back_body(acc_ref, g2_ref, dinv_ref, b2_ref, out_ref):
    dinv = dinv_ref[...]
    acc = acc_ref[:, :N, :]
    out_ref[...] = dinv * (acc[0] + acc[1] + g2_ref[...]) + b2_ref[...]


def kernel(x, edge_index, W1, b1, gamma, beta, W2, b2):
    src = edge_index[0]
    dst = edge_index[1]
    dst2 = dst.reshape(NW, NCHUNK, CHUNK)
    zeros = jnp.zeros((NPAD, D), jnp.float32)
    ones_rows = jnp.ones((CHUNK, D), jnp.float32)
    deg_parts = _sc_degree(dst2, ones_rows, zeros)
    g1, dinv = pl.pallas_call(
        _tc_front_body,
        out_shape=[jax.ShapeDtypeStruct((N, D), jnp.float32),
                   jax.ShapeDtypeStruct((N, D), jnp.float32)],
    )(x, W1, deg_parts)
    acc1 = _sc_scatter(g1, src, dst, zeros)
    g2 = pl.pallas_call(
        _tc_mid_body,
        out_shape=jax.ShapeDtypeStruct((N, D), jnp.float32),
    )(acc1, g1, dinv, b1.reshape(1, D), gamma.reshape(1, D),
      beta.reshape(1, D), W2)
    acc2 = _sc_scatter(g2, src, dst, zeros)
    out = pl.pallas_call(
        _tc_back_body,
        out_shape=jax.ShapeDtypeStruct((N, D), jnp.float32),
    )(acc2, g2, dinv, b2.reshape(1, D))
    return out


# speedup vs baseline: 1.0336x; 1.0216x over previous
import functools

import jax
import jax.numpy as jnp
from jax import lax
from jax.experimental import pallas as pl
from jax.experimental.pallas import tpu as pltpu
from jax.experimental.pallas import tpu_sc as plsc

N = 10000
D = 128
E = 320000
NC = 2
NS = 16
NW = NC * NS
EPW = E // NW
CHUNK = 80
NCHUNK = EPW // CHUNK
NPAD = 10240
RPT = NPAD // NS
NBUF = 4

_mesh = plsc.VectorSubcoreMesh(core_axis_name="c", subcore_axis_name="s")


@functools.partial(
    pl.kernel,
    out_type=jax.ShapeDtypeStruct((NC, NPAD, D), jnp.float32),
    mesh=_mesh,
    scratch_types=[
        pltpu.VMEM((NCHUNK, CHUNK), jnp.int32),
        pltpu.VMEM((CHUNK, D), jnp.float32),
        pltpu.VMEM_SHARED((NPAD, D), jnp.float32),
        pltpu.SemaphoreType.DMA,
        pltpu.SemaphoreType.DMA,
    ],
)
def _sc_degree(dst_hbm, ones_hbm, zeros_hbm, out_hbm, didx_v, ones_v,
               acc_sh, sem0, sem1):
    c = lax.axis_index("c")
    s = lax.axis_index("s")
    wid = c * NS + s
    cp1 = pltpu.async_copy(ones_hbm, ones_v, sem0)
    cp2 = pltpu.async_copy(zeros_hbm.at[pl.ds(s * RPT, RPT)],
                           acc_sh.at[pl.ds(s * RPT, RPT)], sem1)
    pltpu.sync_copy(dst_hbm.at[wid], didx_v)
    cp1.wait()
    cp2.wait()
    plsc.subcore_barrier()

    sems = (sem0, sem1)
    for b in range(2):
        pltpu.async_copy(ones_v, acc_sh.at[didx_v.at[b]], sems[b], add=True)

    def grp(g, carry):
        for b in range(2):
            i = g * 2 + b
            pltpu.make_async_copy(ones_v, acc_sh.at[didx_v.at[i]],
                                  sems[b]).wait()

            @pl.when(i + 2 < NCHUNK)
            def _():
                pltpu.async_copy(ones_v, acc_sh.at[didx_v.at[i + 2]],
                                 sems[b], add=True)
        return carry

    lax.fori_loop(0, NCHUNK // 2, grp, 0)
    pltpu.make_async_copy(ones_v, acc_sh.at[didx_v.at[NCHUNK - 1]],
                          sems[(NCHUNK - 1) % 2]).wait()
    plsc.subcore_barrier()
    pltpu.sync_copy(acc_sh.at[pl.ds(s * RPT, RPT)],
                    out_hbm.at[c, pl.ds(s * RPT, RPT)])


@functools.partial(
    pl.kernel,
    out_type=jax.ShapeDtypeStruct((NC, NPAD, D), jnp.float32),
    mesh=_mesh,
    scratch_types=[
        pltpu.VMEM((2 * NBUF, CHUNK), jnp.int32),
        pltpu.VMEM((NBUF, CHUNK, D), jnp.float32),
        pltpu.VMEM_SHARED((NPAD, D), jnp.float32),
        pltpu.SemaphoreType.DMA,
        pltpu.SemaphoreType.DMA,
        pltpu.SemaphoreType.DMA,
        pltpu.SemaphoreType.DMA,
        pltpu.SemaphoreType.DMA,
        pltpu.SemaphoreType.DMA,
        pltpu.SemaphoreType.DMA,
        pltpu.SemaphoreType.DMA,
        pltpu.SemaphoreType.DMA,
    ],
)
def _sc_scatter(g_hbm, src_hbm, dst_hbm, zeros_hbm, out_hbm,
                idx_v, rows_v, acc_sh,
                isem0, isem1, isem2, isem3, gsem0, gsem1, gsem2, gsem3,
                zsem):
    c = lax.axis_index("c")
    s = lax.axis_index("s")
    wid = c * NS + s
    isems = (isem0, isem1, isem2, isem3)
    gsems = (gsem0, gsem1, gsem2, gsem3)
    ebase = wid * EPW

    zcp = pltpu.async_copy(zeros_hbm.at[pl.ds(s * RPT, RPT)],
                           acc_sh.at[pl.ds(s * RPT, RPT)], zsem)

    def fire_idx(j, b):
        off = ebase + j * CHUNK
        pltpu.async_copy(src_hbm.at[pl.ds(off, CHUNK)], idx_v.at[2 * b],
                         isems[b])
        pltpu.async_copy(dst_hbm.at[pl.ds(off, CHUNK)], idx_v.at[2 * b + 1],
                         isems[b])

    def wait_idx(b):
        pltpu.make_async_copy(src_hbm.at[pl.ds(0, CHUNK)], idx_v.at[2 * b],
                              isems[b]).wait()
        pltpu.make_async_copy(src_hbm.at[pl.ds(0, CHUNK)],
                              idx_v.at[2 * b + 1], isems[b]).wait()

    def fire_gather(b):
        pltpu.async_copy(g_hbm.at[idx_v.at[2 * b]], rows_v.at[b], gsems[b])

    def wait_gather(b):
        pltpu.make_async_copy(g_hbm.at[idx_v.at[2 * b]], rows_v.at[b],
                              gsems[b]).wait()

    for j in range(NBUF - 1):
        fire_idx(j, j)
    for j in range(NBUF - 2):
        wait_idx(j)
        fire_gather(j)
    zcp.wait()
    plsc.subcore_barrier()

    def grp(g, carry):
        for b in range(NBUF):
            i = g * NBUF + b

            wait_gather(b)
            pltpu.sync_copy(rows_v.at[b], acc_sh.at[idx_v.at[2 * b + 1]],
                            add=True)

            @pl.when(i + NBUF - 1 < NCHUNK)
            def _():
                fire_idx(i + NBUF - 1, (b + NBUF - 1) % NBUF)

            @pl.when(i + NBUF - 2 < NCHUNK)
            def _():
                nb = (b + NBUF - 2) % NBUF
                wait_idx(nb)
                fire_gather(nb)
        return carry

    lax.fori_loop(0, NCHUNK // NBUF, grp, 0)
    tb = (NCHUNK - 1) % NBUF
    wait_gather(tb)
    pltpu.sync_copy(rows_v.at[tb], acc_sh.at[idx_v.at[2 * tb + 1]], add=True)

    plsc.subcore_barrier()
    pltpu.sync_copy(acc_sh.at[pl.ds(s * RPT, RPT)],
                    out_hbm.at[c, pl.ds(s * RPT, RPT)])


def _dinv_from_parts(degp):
    deg = degp[0, :N, 0:1] + degp[1, :N, 0:1] + 1.0
    return lax.rsqrt(deg)


def _tc_front_body(x_ref, w1_ref, degp_ref, g_ref, dinv_ref):
    dinv = _dinv_from_parts(degp_ref[...])
    h = jnp.dot(x_ref[...], w1_ref[...], preferred_element_type=jnp.float32)
    g_ref[...] = h * dinv
    dinv_ref[...] = jnp.broadcast_to(dinv, (N, D))


def _tc_mid_body(acc_ref, g1_ref, dinv_ref, b1_ref, gamma_ref, beta_ref,
                 w2_ref, g2_ref):
    dinv = dinv_ref[...]
    acc = acc_ref[:, :N, :]
    h1 = dinv * (acc[0] + acc[1] + g1_ref[...]) + b1_ref[...]
    mean = jnp.mean(h1, axis=0, keepdims=True)
    var = jnp.mean((h1 - mean) ** 2, axis=0, keepdims=True)
    hn = gamma_ref[...] * (h1 - mean) * lax.rsqrt(var + 1e-5) + beta_ref[...]
    hn = jnp.where(hn > 0, hn, 0.01 * hn)
    h2 = jnp.dot(hn, w2_ref[...], preferred_element_type=jnp.float32)
    g2_ref[...] = h2 * dinv


def _tc_back_body(acc_ref, g2_ref, dinv_ref, b2_ref, out_ref):
    dinv = dinv_ref[...]
    acc = acc_ref[:, :N, :]
    out_ref[...] = dinv * (acc[0] + acc[1] + g2_ref[...]) + b2_ref[...]


def kernel(x, edge_index, W1, b1, gamma, beta, W2, b2):
    src = edge_index[0]
    dst = edge_index[1]
    dst2 = dst.reshape(NW, NCHUNK, CHUNK)
    zeros = jnp.zeros((NPAD, D), jnp.float32)
    ones_rows = jnp.ones((CHUNK, D), jnp.float32)
    deg_parts = _sc_degree(dst2, ones_rows, zeros)
    g1, dinv = pl.pallas_call(
        _tc_front_body,
        out_shape=[jax.ShapeDtypeStruct((N, D), jnp.float32),
                   jax.ShapeDtypeStruct((N, D), jnp.float32)],
    )(x, W1, deg_parts)
    acc1 = _sc_scatter(g1, src, dst, zeros)
    g2 = pl.pallas_call(
        _tc_mid_body,
        out_shape=jax.ShapeDtypeStruct((N, D), jnp.float32),
    )(acc1, g1, dinv, b1.reshape(1, D), gamma.reshape(1, D),
      beta.reshape(1, D), W2)
    acc2 = _sc_scatter(g2, src, dst, zeros)
    out = pl.pallas_call(
        _tc_back_body,
        out_shape=jax.ShapeDtypeStruct((N, D), jnp.float32),
    )(acc2, g2, dinv, b2.reshape(1, D))
    return out
